# baseline (speedup 1.0000x reference)
import functools

import jax
import jax.numpy as jnp
from jax.experimental import pallas as pl


def _rowsum_body(a_ref, d_ref):
    j = pl.program_id(1)
    s = jnp.sum(a_ref[...], axis=1, keepdims=True)

    @pl.when(j == 0)
    def _init():
        d_ref[...] = s

    @pl.when(j != 0)
    def _acc():
        d_ref[...] = d_ref[...] + s


def _dinv(d_raw):
    d = d_raw + 1.0
    d = jnp.where(d <= 1e-4, jnp.float32(1e-4), d)
    return jax.lax.rsqrt(d)


def _gcn_body(di_ref, dj_ref, xi_ref, xj_ref, w_ref, a_ref, o_ref, *, c_in):
    j = pl.program_id(1)
    nj = pl.num_programs(1)

    w = w_ref[...]
    dinv_j = _dinv(dj_ref[...])
    xj = xj_ref[...]
    yj = jnp.dot(xj.reshape(-1, c_in), w,
                 preferred_element_type=jnp.float32).reshape(xj.shape[0], -1)
    part = jnp.dot(a_ref[...], dinv_j * yj,
                   preferred_element_type=jnp.float32)

    @pl.when(j == 0)
    def _init():
        dinv_i = _dinv(di_ref[...])
        xi = xi_ref[...]
        yi = jnp.dot(xi.reshape(-1, c_in), w,
                     preferred_element_type=jnp.float32).reshape(xi.shape[0], -1)
        o_ref[...] = dinv_i * yi + part

    @pl.when(j != 0)
    def _acc():
        o_ref[...] = o_ref[...] + part

    @pl.when(j == nj - 1)
    def _fin():
        o_ref[...] = o_ref[...] * _dinv(di_ref[...])


def kernel(X, A, W):
    B, N, C_IN = X.shape
    C_OUT = W.shape[1]

    BI, BJ = 512, 2048
    ni, nj = pl.cdiv(N, BI), pl.cdiv(N, BJ)

    Xr = jnp.transpose(X, (1, 0, 2)).reshape(N, B * C_IN)

    D = pl.pallas_call(
        _rowsum_body,
        grid=(ni, nj),
        in_specs=[pl.BlockSpec((BI, BJ), lambda i, j: (i, j))],
        out_specs=pl.BlockSpec((BI, 1), lambda i, j: (i, 0)),
        out_shape=jax.ShapeDtypeStruct((N, 1), jnp.float32),
    )(A)

    out2 = pl.pallas_call(
        functools.partial(_gcn_body, c_in=C_IN),
        grid=(ni, nj),
        in_specs=[
            pl.BlockSpec((BI, 1), lambda i, j: (i, 0)),
            pl.BlockSpec((BJ, 1), lambda i, j: (j, 0)),
            pl.BlockSpec((BI, B * C_IN), lambda i, j: (i, 0)),
            pl.BlockSpec((BJ, B * C_IN), lambda i, j: (j, 0)),
            pl.BlockSpec((C_IN, C_OUT), lambda i, j: (0, 0)),
            pl.BlockSpec((BI, BJ), lambda i, j: (i, j)),
        ],
        out_specs=pl.BlockSpec((BI, B * C_OUT), lambda i, j: (i, 0)),
        out_shape=jax.ShapeDtypeStruct((N, B * C_OUT), jnp.float32),
    )(D, D, Xr, Xr, W, A)

    return out2.reshape(N, B, C_OUT).transpose(1, 0, 2)


# speedup vs baseline: 1.0458x; 1.0458x over previous
import functools

import jax
import jax.numpy as jnp
from jax.experimental import pallas as pl
from jax.experimental.pallas import tpu as pltpu


def _dinv_from_rowsum(s):
    d = s + 1.0
    d = jnp.where(d <= 1e-4, jnp.float32(1e-4), d)
    return jax.lax.rsqrt(d)


def _prep_body(a_ref, x_ref, w_ref, z_ref, e_ref, dinv_ref, *, c_in):
    s = jnp.sum(a_ref[...], axis=1, keepdims=True)
    dinv = _dinv_from_rowsum(s)
    x = x_ref[...]
    y = jnp.dot(x.reshape(-1, c_in), w_ref[...],
                preferred_element_type=jnp.float32).reshape(x.shape[0], -1)
    z = dinv * y
    z_ref[...] = z
    e_ref[...] = dinv * z
    dinv_ref[...] = dinv


def _mm_body(dinv_ref, e_ref, z_ref, a_ref, o_ref):
    j = pl.program_id(1)
    nj = pl.num_programs(1)

    part = jnp.dot(a_ref[...], z_ref[...], preferred_element_type=jnp.float32)

    @pl.when(j == 0)
    def _init():
        o_ref[...] = part

    @pl.when(j != 0)
    def _acc():
        o_ref[...] = o_ref[...] + part

    @pl.when(j == nj - 1)
    def _fin():
        o_ref[...] = o_ref[...] * dinv_ref[...] + e_ref[...]


def kernel(X, A, W):
    B, N, C_IN = X.shape
    C_OUT = W.shape[1]
    F = B * C_OUT

    Xr = jnp.transpose(X, (1, 0, 2)).reshape(N, B * C_IN)

    BP = 256
    Z, E, Dinv = pl.pallas_call(
        functools.partial(_prep_body, c_in=C_IN),
        grid=(N // BP,),
        in_specs=[
            pl.BlockSpec((BP, N), lambda i: (i, 0)),
            pl.BlockSpec((BP, B * C_IN), lambda i: (i, 0)),
            pl.BlockSpec((C_IN, C_OUT), lambda i: (0, 0)),
        ],
        out_specs=[
            pl.BlockSpec((BP, F), lambda i: (i, 0)),
            pl.BlockSpec((BP, F), lambda i: (i, 0)),
            pl.BlockSpec((BP, 1), lambda i: (i, 0)),
        ],
        out_shape=[
            jax.ShapeDtypeStruct((N, F), jnp.float32),
            jax.ShapeDtypeStruct((N, F), jnp.float32),
            jax.ShapeDtypeStruct((N, 1), jnp.float32),
        ],
        compiler_params=pltpu.CompilerParams(
            dimension_semantics=("arbitrary",),
        ),
    )(A, Xr, W)

    BI, BJ = 512, 2048
    ni, nj = N // BI, N // BJ
    out2 = pl.pallas_call(
        _mm_body,
        grid=(ni, nj),
        in_specs=[
            pl.BlockSpec((BI, 1), lambda i, j: (i, 0)),
            pl.BlockSpec((BI, F), lambda i, j: (i, 0)),
            pl.BlockSpec((BJ, F), lambda i, j: (j, 0)),
            pl.BlockSpec((BI, BJ), lambda i, j: (i, j)),
        ],
        out_specs=pl.BlockSpec((BI, F), lambda i, j: (i, 0)),
        out_shape=jax.ShapeDtypeStruct((N, F), jnp.float32),
        compiler_params=pltpu.CompilerParams(
            dimension_semantics=("parallel", "arbitrary"),
        ),
    )(Dinv, E, Z, A)

    return out2.reshape(N, B, C_OUT).transpose(1, 0, 2)


# speedup vs baseline: 1.1858x; 1.1339x over previous
import functools

import jax
import jax.numpy as jnp
from jax.experimental import pallas as pl
from jax.experimental.pallas import tpu as pltpu


def _dinv_from_rowsum(s):
    d = s + 1.0
    d = jnp.where(d <= 1e-4, jnp.float32(1e-4), d)
    return jax.lax.rsqrt(d)


def _prep_body(a_ref, x_ref, w_ref, z_ref, e_ref, dinv_ref, *, c_in):
    s = jnp.sum(a_ref[...], axis=1, keepdims=True)
    dinv = _dinv_from_rowsum(s)
    x = x_ref[...]
    y = jnp.dot(x.reshape(-1, c_in), w_ref[...],
                preferred_element_type=jnp.float32).reshape(x.shape[0], -1)
    z = dinv * y
    z_ref[...] = z
    e_ref[...] = dinv * z
    dinv_ref[...] = dinv


def _mm_body(dinv_ref, e_ref, z_ref, a_ref, o_ref):
    j = pl.program_id(1)
    nj = pl.num_programs(1)

    part = jnp.dot(a_ref[...], z_ref[...], preferred_element_type=jnp.float32)

    @pl.when(j == 0)
    def _init():
        o_ref[...] = part

    @pl.when(j != 0)
    def _acc():
        o_ref[...] = o_ref[...] + part

    @pl.when(j == nj - 1)
    def _fin():
        o_ref[...] = o_ref[...] * dinv_ref[...] + e_ref[...]


def kernel(X, A, W):
    B, N, C_IN = X.shape
    C_OUT = W.shape[1]
    F = B * C_OUT

    Xr = jnp.transpose(X, (1, 0, 2)).reshape(N, B * C_IN)

    BP = 512
    Z, E, Dinv = pl.pallas_call(
        functools.partial(_prep_body, c_in=C_IN),
        grid=(N // BP,),
        in_specs=[
            pl.BlockSpec((BP, N), lambda i: (i, 0)),
            pl.BlockSpec((BP, B * C_IN), lambda i: (i, 0)),
            pl.BlockSpec((C_IN, C_OUT), lambda i: (0, 0)),
        ],
        out_specs=[
            pl.BlockSpec((BP, F), lambda i: (i, 0)),
            pl.BlockSpec((BP, F), lambda i: (i, 0)),
            pl.BlockSpec((BP, 1), lambda i: (i, 0)),
        ],
        out_shape=[
            jax.ShapeDtypeStruct((N, F), jnp.float32),
            jax.ShapeDtypeStruct((N, F), jnp.float32),
            jax.ShapeDtypeStruct((N, 1), jnp.float32),
        ],
        compiler_params=pltpu.CompilerParams(
            dimension_semantics=("arbitrary",),
        ),
    )(A, Xr, W)

    BI, BJ = 512, 8192
    ni, nj = N // BI, N // BJ
    out2 = pl.pallas_call(
        _mm_body,
        grid=(ni, nj),
        in_specs=[
            pl.BlockSpec((BI, 1), lambda i, j: (i, 0)),
            pl.BlockSpec((BI, F), lambda i, j: (i, 0)),
            pl.BlockSpec((BJ, F), lambda i, j: (j, 0)),
            pl.BlockSpec((BI, BJ), lambda i, j: (i, j)),
        ],
        out_specs=pl.BlockSpec((BI, F), lambda i, j: (i, 0)),
        out_shape=jax.ShapeDtypeStruct((N, F), jnp.float32),
        compiler_params=pltpu.CompilerParams(
            dimension_semantics=("parallel", "arbitrary"),
        ),
    )(Dinv, E, Z, A)

    return out2.reshape(N, B, C_OUT).transpose(1, 0, 2)


# speedup vs baseline: 1.2894x; 1.0874x over previous
import functools

import jax
import jax.numpy as jnp
from jax.experimental import pallas as pl
from jax.experimental.pallas import tpu as pltpu


def _dinv_from_rowsum(s):
    d = s + 1.0
    d = jnp.where(d <= 1e-4, jnp.float32(1e-4), d)
    return jax.lax.rsqrt(d)


def _sweep_body(x_ref, w_ref, a_ref, p_ref, dinv_ref, z_ref, zsc, *, c_in, br, n):
    r = pl.program_id(0)
    nr = pl.num_programs(0)

    a = a_ref[...]
    s = jnp.sum(a, axis=1, keepdims=True)
    dinv = _dinv_from_rowsum(s)
    x = x_ref[...]
    y = jnp.dot(x.reshape(-1, c_in), w_ref[...],
                preferred_element_type=jnp.float32).reshape(x.shape[0], -1)
    z = dinv * y
    dinv_ref[...] = dinv
    z_ref[...] = z
    zsc[pl.ds(r * br, br), :] = z

    row_ids = jax.lax.broadcasted_iota(jnp.int32, (n, zsc.shape[1]), 0)
    zfull = jnp.where(row_ids < (r + 1) * br, zsc[...], 0.0)
    acc = jnp.dot(a, zfull, preferred_element_type=jnp.float32)

    @pl.when(r == nr - 1)
    def _finalize_last():
        p_ref[...] = acc * dinv + dinv * z

    @pl.when(r != nr - 1)
    def _partial():
        p_ref[...] = acc


def kernel(X, A, W):
    B, N, C_IN = X.shape
    C_OUT = W.shape[1]
    F = B * C_OUT

    BR = 512
    BC = 2048
    nr = N // BR
    ncb = N // BC

    fb = [((r + 1) * BR) // BC for r in range(nr)]
    cnt = [ncb - fb[r] for r in range(nr)]
    off = [0] * (nr + 1)
    for r in range(nr):
        off[r + 1] = off[r] + cnt[r]
    nsteps = off[nr]

    def row_of(k):
        r = jnp.int32(0)
        for t in range(1, nr):
            r = r + jnp.where(k >= off[t], 1, 0).astype(jnp.int32)
        return r

    def off_of(r):
        o = jnp.int32(0)
        for t in range(nr - 1):
            o = o + jnp.where(r > t, cnt[t], 0).astype(jnp.int32)
        return o

    def fb_of(r):
        return ((r + 1) * BR) // BC

    def colblk_of(k):
        r = row_of(k)
        return fb_of(r) + (k - off_of(r))

    Xr = jnp.transpose(X, (1, 0, 2)).reshape(N, B * C_IN)

    def sweep(x_ref, w_ref, a_ref, p_ref, dinv_ref, z_ref, zsc):
        _sweep_body(x_ref, w_ref, a_ref, p_ref, dinv_ref, z_ref, zsc,
                    c_in=C_IN, br=BR, n=N)

    P, Dinv, Z = pl.pallas_call(
        sweep,
        grid=(nr,),
        in_specs=[
            pl.BlockSpec((BR, B * C_IN), lambda r: (r, 0)),
            pl.BlockSpec((C_IN, C_OUT), lambda r: (0, 0)),
            pl.BlockSpec((BR, N), lambda r: (r, 0)),
        ],
        out_specs=[
            pl.BlockSpec((BR, F), lambda r: (r, 0)),
            pl.BlockSpec((BR, 1), lambda r: (r, 0)),
            pl.BlockSpec((BR, F), lambda r: (r, 0)),
        ],
        out_shape=[
            jax.ShapeDtypeStruct((N, F), jnp.float32),
            jax.ShapeDtypeStruct((N, 1), jnp.float32),
            jax.ShapeDtypeStruct((N, F), jnp.float32),
        ],
        scratch_shapes=[pltpu.VMEM((N, F), jnp.float32)],
        compiler_params=pltpu.CompilerParams(
            dimension_semantics=("arbitrary",),
        ),
    )(Xr, W, A)

    def upper(p_ref, dinv_ref, zr_ref, zc_ref, a_ref, o_ref):
        k = pl.program_id(0)
        r = row_of(k)
        cb = fb_of(r) + (k - off_of(r))
        off_r = off_of(r)
        cnt_r = ncb - fb_of(r)

        zc = zc_ref[...]
        col_ids = jax.lax.broadcasted_iota(jnp.int32, zc.shape, 0) + cb * BC
        zm = jnp.where(col_ids >= (r + 1) * BR, zc, 0.0)
        part = jnp.dot(a_ref[...], zm, preferred_element_type=jnp.float32)

        @pl.when(k == off_r)
        def _first():
            o_ref[...] = p_ref[...] + part

        @pl.when(k != off_r)
        def _acc():
            o_ref[...] = o_ref[...] + part

        @pl.when(k == off_r + cnt_r - 1)
        def _last():
            dinv = dinv_ref[...]
            o_ref[...] = o_ref[...] * dinv + dinv * zr_ref[...]

    Ofull = pl.pallas_call(
        upper,
        grid=(nsteps,),
        in_specs=[
            pl.BlockSpec((BR, F), lambda k: (row_of(k), 0)),
            pl.BlockSpec((BR, 1), lambda k: (row_of(k), 0)),
            pl.BlockSpec((BR, F), lambda k: (row_of(k), 0)),
            pl.BlockSpec((BC, F), lambda k: (colblk_of(k), 0)),
            pl.BlockSpec((BR, BC), lambda k: (row_of(k), colblk_of(k))),
        ],
        out_specs=pl.BlockSpec((BR, F), lambda k: (row_of(k), 0)),
        out_shape=jax.ShapeDtypeStruct((N, F), jnp.float32),
        compiler_params=pltpu.CompilerParams(
            dimension_semantics=("arbitrary",),
        ),
    )(P, Dinv, Z, Z, A)

    out2 = jnp.concatenate([Ofull[: (nr - 1) * BR], P[(nr - 1) * BR:]], axis=0)
    return out2.reshape(N, B, C_OUT).transpose(1, 0, 2)
